# baseline (speedup 1.0000x reference)
import functools

import jax
import jax.numpy as jnp
from jax import lax
from jax.experimental import pallas as pl
from jax.experimental.pallas import tpu as pltpu
from jax.experimental.pallas import tpu_sc as plsc

VOCAB = 1000000
MAXLEN = 200
EMBED_DIM = 64
BATCH = 4096

NC = 2
NS = 16
NW = NC * NS
TOKENS = BATCH * MAXLEN
PER_W = TOKENS // NW
CHUNK = 128
N_CHUNKS = PER_W // CHUNK
LANES = 16


def _body(idx_hbm, table_hbm, pos_hbm, out_hbm, idx_v, rows_v, pos2_v, sem):
    wid = lax.axis_index("s") * NC + lax.axis_index("c")
    base0 = wid * PER_W

    pltpu.sync_copy(pos_hbm, pos2_v.at[pl.ds(0, MAXLEN)])
    pltpu.sync_copy(pos_hbm, pos2_v.at[pl.ds(MAXLEN, MAXLEN)])

    def chunk_body(k, carry):
        base = base0 + k * CHUNK
        pltpu.sync_copy(idx_hbm.at[pl.ds(base, CHUNK)], idx_v)
        pltpu.async_copy(table_hbm.at[idx_v], rows_v, sem).wait()
        ph = lax.rem(k * CHUNK, MAXLEN)

        def row_body(j, c2):
            pj = ph + j
            for c in range(EMBED_DIM // LANES):
                sl = pl.ds(c * LANES, LANES)
                rows_v[j, sl] = rows_v[j, sl] + pos2_v[pj, sl]
            return c2

        lax.fori_loop(0, CHUNK, row_body, 0)
        pltpu.sync_copy(rows_v, out_hbm.at[pl.ds(base, CHUNK)])
        return carry

    lax.fori_loop(0, N_CHUNKS, chunk_body, 0)


def kernel(inputs, token_table, pos_table):
    idx = jnp.reshape(inputs, (TOKENS,)).astype(jnp.int32)
    mesh = plsc.VectorSubcoreMesh(core_axis_name="c", subcore_axis_name="s")
    fn = functools.partial(
        pl.kernel,
        mesh=mesh,
        compiler_params=pltpu.CompilerParams(use_tc_tiling_on_sc=False),
        out_type=jax.ShapeDtypeStruct((TOKENS, EMBED_DIM), jnp.float32),
        scratch_types=[
            pltpu.VMEM((CHUNK,), jnp.int32),
            pltpu.VMEM((CHUNK, EMBED_DIM), jnp.float32),
            pltpu.VMEM((2 * MAXLEN, EMBED_DIM), jnp.float32),
            pltpu.SemaphoreType.DMA,
        ],
    )(_body)
    out = fn(idx, token_table, pos_table)
    return jnp.reshape(out, (BATCH, MAXLEN, EMBED_DIM))


# speedup vs baseline: 1.1724x; 1.1724x over previous
import functools

import jax
import jax.numpy as jnp
from jax import lax
from jax.experimental import pallas as pl
from jax.experimental.pallas import tpu as pltpu
from jax.experimental.pallas import tpu_sc as plsc

VOCAB = 1000000
MAXLEN = 200
EMBED_DIM = 64
BATCH = 4096

NC = 2
NS = 16
NW = NC * NS
TOKENS = BATCH * MAXLEN
PER_W = TOKENS // NW
CHUNK = 128
N_CHUNKS = PER_W // CHUNK
LANES = 16
NBUF = 4


def _body(idx_hbm, table_hbm, pos_hbm, out_hbm, idx_v, rows_v, pos2_v, *sems):
    gsems = sems[:NBUF]
    osems = sems[NBUF:]
    wid = lax.axis_index("s") * NC + lax.axis_index("c")
    base0 = wid * PER_W

    pltpu.sync_copy(idx_hbm.at[pl.ds(base0, PER_W)], idx_v)
    pltpu.sync_copy(pos_hbm, pos2_v.at[pl.ds(0, MAXLEN)])
    pltpu.sync_copy(pos_hbm, pos2_v.at[pl.ds(MAXLEN, MAXLEN)])

    def gather(i, s):
        off = pl.multiple_of(i * CHUNK, CHUNK)
        return pltpu.make_async_copy(
            table_hbm.at[idx_v.at[pl.ds(off, CHUNK)]],
            rows_v.at[s],
            gsems[s])

    def out_copy(i, s):
        off = pl.multiple_of(base0 + i * CHUNK, CHUNK)
        return pltpu.make_async_copy(
            rows_v.at[s],
            out_hbm.at[pl.ds(off, CHUNK)],
            osems[s])

    for s in range(NBUF - 1):
        gather(s, s).start()

    def chunk_body(i0, carry):
        for s in range(NBUF):
            i = i0 * NBUF + s
            sp = (s + NBUF - 1) % NBUF
            pf = i + NBUF - 1

            @pl.when(i > 0)
            def _():
                out_copy(i - 1, sp).wait()

            @pl.when(pf < N_CHUNKS)
            def _():
                gather(pf, sp).start()

            gather(i, s).wait()

            ph = lax.rem(i * CHUNK, MAXLEN)

            def row_body(jj, c2):
                j = jj * 2
                for r in range(2):
                    for c in range(EMBED_DIM // LANES):
                        sl = pl.ds(c * LANES, LANES)
                        rows_v[s, j + r, sl] = (
                            rows_v[s, j + r, sl] + pos2_v[ph + j + r, sl])
                return c2

            lax.fori_loop(0, CHUNK // 2, row_body, 0)
            out_copy(i, s).start()
        return carry

    lax.fori_loop(0, N_CHUNKS // NBUF, chunk_body, 0)
    out_copy(N_CHUNKS - 1, (N_CHUNKS - 1) % NBUF).wait()


def kernel(inputs, token_table, pos_table):
    idx = jnp.reshape(inputs, (TOKENS,)).astype(jnp.int32)
    mesh = plsc.VectorSubcoreMesh(core_axis_name="c", subcore_axis_name="s")
    fn = functools.partial(
        pl.kernel,
        mesh=mesh,
        compiler_params=pltpu.CompilerParams(use_tc_tiling_on_sc=False),
        out_type=jax.ShapeDtypeStruct((TOKENS, EMBED_DIM), jnp.float32),
        scratch_types=[
            pltpu.VMEM((PER_W,), jnp.int32),
            pltpu.VMEM((NBUF, CHUNK, EMBED_DIM), jnp.float32),
            pltpu.VMEM((2 * MAXLEN, EMBED_DIM), jnp.float32),
        ] + [pltpu.SemaphoreType.DMA] * (2 * NBUF),
    )(_body)
    out = fn(idx, token_table, pos_table)
    return jnp.reshape(out, (BATCH, MAXLEN, EMBED_DIM))


# speedup vs baseline: 1.5539x; 1.3255x over previous
import functools

import jax
import jax.numpy as jnp
from jax import lax
from jax.experimental import pallas as pl
from jax.experimental.pallas import tpu as pltpu
from jax.experimental.pallas import tpu_sc as plsc

VOCAB = 1000000
MAXLEN = 200
EMBED_DIM = 64
BATCH = 4096

NC = 2
NS = 16
NW = NC * NS
TOKENS = BATCH * MAXLEN
PER_W = TOKENS // NW
CHUNK = MAXLEN
SUB = (128, 72)
N_CHUNKS = PER_W // CHUNK
LANES = 16
NBUF = 4
ROW_UNROLL = 8


def _body(idx_hbm, table_hbm, pos_hbm, out_hbm, idx_v, rows_v, pos_v, *sems):
    gsems = sems[:NBUF]
    osems = sems[NBUF:]
    wid = lax.axis_index("s") * NC + lax.axis_index("c")
    base0 = wid * PER_W

    pltpu.sync_copy(idx_hbm.at[pl.ds(base0, PER_W)], idx_v)
    pltpu.sync_copy(pos_hbm, pos_v)

    def gathers(i, s):
        off = pl.multiple_of(i * CHUNK, CHUNK)
        cs = []
        sub_off = 0
        for n in SUB:
            cs.append(pltpu.make_async_copy(
                table_hbm.at[idx_v.at[pl.ds(off + sub_off, n)]],
                rows_v.at[s, pl.ds(sub_off, n)],
                gsems[s]))
            sub_off += n
        return cs

    def out_copy(i, s):
        off = pl.multiple_of(base0 + i * CHUNK, CHUNK)
        return pltpu.make_async_copy(
            rows_v.at[s],
            out_hbm.at[pl.ds(off, CHUNK)],
            osems[s])

    for s in range(NBUF - 1):
        for c in gathers(s, s):
            c.start()

    def chunk_body(i0, carry):
        for s in range(NBUF):
            i = i0 * NBUF + s
            sp = (s + NBUF - 1) % NBUF
            pf = i + NBUF - 1

            @pl.when(i > 0)
            def _():
                out_copy(i - 1, sp).wait()

            @pl.when(pf < N_CHUNKS)
            def _():
                for c in gathers(pf, sp):
                    c.start()

            for c in gathers(i, s):
                c.wait()

            def row_body(jj, c2):
                j = jj * ROW_UNROLL
                for r in range(ROW_UNROLL):
                    for c in range(EMBED_DIM // LANES):
                        sl = pl.ds(c * LANES, LANES)
                        rows_v[s, j + r, sl] = (
                            rows_v[s, j + r, sl] + pos_v[j + r, sl])
                return c2

            lax.fori_loop(0, CHUNK // ROW_UNROLL, row_body, 0)
            out_copy(i, s).start()
        return carry

    lax.fori_loop(0, N_CHUNKS // NBUF, chunk_body, 0)
    out_copy(N_CHUNKS - 1, (N_CHUNKS - 1) % NBUF).wait()


def kernel(inputs, token_table, pos_table):
    idx = jnp.reshape(inputs, (TOKENS,)).astype(jnp.int32)
    mesh = plsc.VectorSubcoreMesh(core_axis_name="c", subcore_axis_name="s")
    fn = functools.partial(
        pl.kernel,
        mesh=mesh,
        compiler_params=pltpu.CompilerParams(use_tc_tiling_on_sc=False),
        out_type=jax.ShapeDtypeStruct((TOKENS, EMBED_DIM), jnp.float32),
        scratch_types=[
            pltpu.VMEM((PER_W,), jnp.int32),
            pltpu.VMEM((NBUF, CHUNK, EMBED_DIM), jnp.float32),
            pltpu.VMEM((MAXLEN, EMBED_DIM), jnp.float32),
        ] + [pltpu.SemaphoreType.DMA] * (2 * NBUF),
    )(_body)
    out = fn(idx, token_table, pos_table)
    return jnp.reshape(out, (BATCH, MAXLEN, EMBED_DIM))
